# baseline (speedup 1.0000x reference)
import jax
import jax.numpy as jnp
from jax import lax
from jax.experimental import pallas as pl
from jax.experimental.pallas import tpu as pltpu
from jax.experimental.pallas import tpu_sc as plsc

N_NODES = 10000
N_EDGES = 160000
D = 256
HALF = 128
EPS1 = 1.5

NUM_SC = 2
NUM_SUBCORES = 16
BATCH = 128
NB = 80
CHUNK = 16
NCH = NB // CHUNK
E_PER_TILE = NB * BATCH
E_PAD = NUM_SUBCORES * E_PER_TILE
ACC_ROWS = 10240
ZSTRIPE = ACC_ROWS // NUM_SUBCORES


def _sc_agg_kernel(xs_hbm, src_hbm, dst_hbm, out_hbm, acc, src_v, dst_v,
                   rows_a, rows_b, sem_a, sem_b):
    c = lax.axis_index("c")
    s = lax.axis_index("s")

    zeros16 = jnp.zeros((16,), jnp.float32)

    @pl.loop(0, BATCH)
    def _(i):
        @pl.loop(0, HALF // 16)
        def _(k):
            rows_a[i, pl.ds(k * 16, 16)] = zeros16

    for k in range(ZSTRIPE // BATCH):
        pltpu.sync_copy(rows_a, acc.at[pl.ds(s * ZSTRIPE + k * BATCH, BATCH)])

    plsc.subcore_barrier()

    def gather(j, buf, sem):
        return pltpu.make_async_copy(xs_hbm.at[src_v.at[j]], buf, sem)

    def add(j, buf):
        pltpu.sync_copy(buf, acc.at[dst_v.at[j]], add=True)

    for g in range(NCH):
        pltpu.sync_copy(src_hbm.at[c, s, pl.ds(g * CHUNK, CHUNK)], src_v)
        pltpu.sync_copy(dst_hbm.at[s, pl.ds(g * CHUNK, CHUNK)], dst_v)

        gather(0, rows_a, sem_a).start()

        @pl.loop(0, CHUNK - 2, step=2)
        def _(j):
            gather(j, rows_a, sem_a).wait()
            gather(j + 1, rows_b, sem_b).start()
            add(j, rows_a)
            gather(j + 1, rows_b, sem_b).wait()
            gather(j + 2, rows_a, sem_a).start()
            add(j + 1, rows_b)

        gather(CHUNK - 2, rows_a, sem_a).wait()
        gather(CHUNK - 1, rows_b, sem_b).start()
        add(CHUNK - 2, rows_a)
        gather(CHUNK - 1, rows_b, sem_b).wait()
        add(CHUNK - 1, rows_b)

    plsc.subcore_barrier()

    pltpu.sync_copy(acc.at[pl.ds(s * ZSTRIPE, ZSTRIPE)],
                    out_hbm.at[c, pl.ds(s * ZSTRIPE, ZSTRIPE)])


def _sc_aggregate(xs, srcs, dsts):
    mesh = plsc.VectorSubcoreMesh(core_axis_name="c", subcore_axis_name="s")
    kern = pl.kernel(
        _sc_agg_kernel,
        out_type=jax.ShapeDtypeStruct((NUM_SC, ACC_ROWS, HALF), jnp.float32),
        mesh=mesh,
        scratch_types=[
            pltpu.VMEM_SHARED((ACC_ROWS, HALF), jnp.float32),
            pltpu.VMEM((CHUNK, BATCH), jnp.int32),
            pltpu.VMEM((CHUNK, BATCH), jnp.int32),
            pltpu.VMEM((BATCH, HALF), jnp.float32),
            pltpu.VMEM((BATCH, HALF), jnp.float32),
            pltpu.SemaphoreType.DMA,
            pltpu.SemaphoreType.DMA,
        ],
    )
    return kern(xs, srcs, dsts)


def _mm_body(x_ref, lo_ref, hi_ref, w_ref, o_ref):
    agg = jnp.concatenate([lo_ref[0], hi_ref[0]], axis=-1)
    xa = EPS1 * x_ref[...] + agg
    o_ref[...] = jnp.dot(xa, w_ref[...], preferred_element_type=jnp.float32)


def _tc_linear(x, agg_pad, W):
    rows = 1000
    grid = (N_NODES // rows,)
    return pl.pallas_call(
        _mm_body,
        grid=grid,
        in_specs=[
            pl.BlockSpec((rows, D), lambda i: (i, 0)),
            pl.BlockSpec((1, rows, HALF), lambda i: (0, i, 0)),
            pl.BlockSpec((1, rows, HALF), lambda i: (1, i, 0)),
            pl.BlockSpec((D, D), lambda i: (0, 0)),
        ],
        out_specs=pl.BlockSpec((rows, D), lambda i: (i, 0)),
        out_shape=jax.ShapeDtypeStruct((N_NODES, D), jnp.float32),
    )(x, agg_pad, agg_pad, W)


def kernel(x, edge_index, W):
    src = edge_index[0].astype(jnp.int32)
    dst = edge_index[1].astype(jnp.int32)

    pad = E_PAD - N_EDGES
    src_p = jnp.concatenate([src, jnp.zeros((pad,), jnp.int32)])
    dst_p = jnp.concatenate([dst, jnp.full((pad,), N_NODES, jnp.int32)])
    src_t = src_p.reshape(NUM_SUBCORES, NB, BATCH)
    srcs = jnp.stack([src_t, src_t + N_NODES])
    dsts = dst_p.reshape(NUM_SUBCORES, NB, BATCH)

    xs = jnp.concatenate([x[:, :HALF], x[:, HALF:]], axis=0)

    agg_pad = _sc_aggregate(xs, srcs, dsts)
    return _tc_linear(x, agg_pad, W)
